# baseline (speedup 1.0000x reference)
import jax
import jax.numpy as jnp
from jax import lax
from jax.experimental import pallas as pl
from jax.experimental.pallas import tpu as pltpu
from jax.experimental.pallas import tpu_sc as plsc

N = 10000
E = 320000
IN_C = 128
HID = 256

NC = 2
NS = 16
L = 16

NP = 10240
PADN = 10200
EPAD = 327680
G = EPAD // 128
GPT = G // (NC * NS)
BLK = 1024
GRID = NP // BLK

HH = HID // 2
CH = 256
EPC = EPAD // NS
NCH = EPC // CH
RPT = NP // NS
ZR = 32

_SC_MESH = plsc.VectorSubcoreMesh(core_axis_name="c", subcore_axis_name="s")


def _mm1_body(x_ref, w_ref, as_ref, ad_ref,
              h_ref, s_ref, d_ref, sm_ref, dm_ref, mx_ref):
    i = pl.program_id(0)
    h = jnp.dot(x_ref[...], w_ref[...], preferred_element_type=jnp.float32)
    h_ref[0] = h[:, :HH]
    h_ref[1] = h[:, HH:]
    s = jnp.dot(h, as_ref[...], preferred_element_type=jnp.float32)
    d = jnp.dot(h, ad_ref[...], preferred_element_type=jnp.float32)
    s_ref[...] = s
    d_ref[...] = d
    sblk = jnp.max(s)
    dblk = jnp.max(d)

    @pl.when(i == 0)
    def _():
        mx_ref[0, 0] = sblk
        mx_ref[0, 1] = dblk

    @pl.when(i > 0)
    def _():
        mx_ref[0, 0] = jnp.maximum(mx_ref[0, 0], sblk)
        mx_ref[0, 1] = jnp.maximum(mx_ref[0, 1], dblk)

    @pl.when(i == GRID - 1)
    def _():
        sm_ref[...] = jnp.full((1, 1), mx_ref[0, 0], jnp.float32)
        dm_ref[...] = jnp.full((1, 1), mx_ref[0, 1], jnp.float32)


def _mm1(x, w1, a_src, a_dst):
    return pl.pallas_call(
        _mm1_body,
        grid=(GRID,),
        in_specs=[
            pl.BlockSpec((BLK, IN_C), lambda i: (i, 0)),
            pl.BlockSpec((IN_C, HID), lambda i: (0, 0)),
            pl.BlockSpec((HID, 1), lambda i: (0, 0)),
            pl.BlockSpec((HID, 1), lambda i: (0, 0)),
        ],
        out_specs=[
            pl.BlockSpec((2, BLK, HH), lambda i: (0, i, 0)),
            pl.BlockSpec((BLK, 1), lambda i: (i, 0)),
            pl.BlockSpec((BLK, 1), lambda i: (i, 0)),
            pl.BlockSpec((1, 1), lambda i: (0, 0)),
            pl.BlockSpec((1, 1), lambda i: (0, 0)),
        ],
        out_shape=[
            jax.ShapeDtypeStruct((2, NP, HH), jnp.float32),
            jax.ShapeDtypeStruct((NP, 1), jnp.float32),
            jax.ShapeDtypeStruct((NP, 1), jnp.float32),
            jax.ShapeDtypeStruct((1, 1), jnp.float32),
            jax.ShapeDtypeStruct((1, 1), jnp.float32),
        ],
        scratch_shapes=[pltpu.SMEM((1, 2), jnp.float32)],
        compiler_params=pltpu.CompilerParams(
            dimension_semantics=("arbitrary",)),
    )(x, w1, a_src, a_dst)


def _mm2_body(o1_ref, b1_ref, w2_ref, as_ref, ad_ref,
              h_ref, s_ref, d_ref, sm_ref, dm_ref, mx_ref):
    i = pl.program_id(0)
    hr0 = jnp.maximum(o1_ref[0] + b1_ref[:, :HH], 0.0)
    hr1 = jnp.maximum(o1_ref[1] + b1_ref[:, HH:], 0.0)
    h = (jnp.dot(hr0, w2_ref[:HH, :], preferred_element_type=jnp.float32)
         + jnp.dot(hr1, w2_ref[HH:, :], preferred_element_type=jnp.float32))
    h_ref[...] = h
    s = jnp.dot(h, as_ref[...], preferred_element_type=jnp.float32)
    d = jnp.dot(h, ad_ref[...], preferred_element_type=jnp.float32)
    s_ref[...] = s
    d_ref[...] = d
    sblk = jnp.max(s)
    dblk = jnp.max(d)

    @pl.when(i == 0)
    def _():
        mx_ref[0, 0] = sblk
        mx_ref[0, 1] = dblk

    @pl.when(i > 0)
    def _():
        mx_ref[0, 0] = jnp.maximum(mx_ref[0, 0], sblk)
        mx_ref[0, 1] = jnp.maximum(mx_ref[0, 1], dblk)

    @pl.when(i == GRID - 1)
    def _():
        sm_ref[...] = jnp.full((1, 1), mx_ref[0, 0], jnp.float32)
        dm_ref[...] = jnp.full((1, 1), mx_ref[0, 1], jnp.float32)


def _mm2(o1, b1, w2, a_src, a_dst):
    return pl.pallas_call(
        _mm2_body,
        grid=(GRID,),
        in_specs=[
            pl.BlockSpec((2, BLK, HH), lambda i: (0, i, 0)),
            pl.BlockSpec((1, HID), lambda i: (0, 0)),
            pl.BlockSpec((HID, HID), lambda i: (0, 0)),
            pl.BlockSpec((HID, 1), lambda i: (0, 0)),
            pl.BlockSpec((HID, 1), lambda i: (0, 0)),
        ],
        out_specs=[
            pl.BlockSpec((BLK, HID), lambda i: (i, 0)),
            pl.BlockSpec((BLK, 1), lambda i: (i, 0)),
            pl.BlockSpec((BLK, 1), lambda i: (i, 0)),
            pl.BlockSpec((1, 1), lambda i: (0, 0)),
            pl.BlockSpec((1, 1), lambda i: (0, 0)),
        ],
        out_shape=[
            jax.ShapeDtypeStruct((NP, HID), jnp.float32),
            jax.ShapeDtypeStruct((NP, 1), jnp.float32),
            jax.ShapeDtypeStruct((NP, 1), jnp.float32),
            jax.ShapeDtypeStruct((1, 1), jnp.float32),
            jax.ShapeDtypeStruct((1, 1), jnp.float32),
        ],
        scratch_shapes=[pltpu.SMEM((1, 2), jnp.float32)],
        compiler_params=pltpu.CompilerParams(
            dimension_semantics=("arbitrary",)),
    )(o1, b1, w2, a_src, a_dst)


def _pool_body(w_ref, h_ref, b2_ref, o_ref, acc_ref):
    i = pl.program_id(0)
    ws = w_ref[0:1, :] + w_ref[1:2, :]
    p = jnp.dot(ws, h_ref[...], preferred_element_type=jnp.float32)

    @pl.when(i == 0)
    def _():
        acc_ref[...] = p

    @pl.when(i > 0)
    def _():
        acc_ref[...] = acc_ref[...] + p

    @pl.when(i == GRID - 1)
    def _():
        o_ref[...] = acc_ref[...] + jnp.float32(N) * b2_ref[...]


def _pool(w, h2, b2):
    return pl.pallas_call(
        _pool_body,
        grid=(GRID,),
        in_specs=[
            pl.BlockSpec((2, BLK), lambda i: (0, i)),
            pl.BlockSpec((BLK, HID), lambda i: (i, 0)),
            pl.BlockSpec((1, HID), lambda i: (0, 0)),
        ],
        out_specs=pl.BlockSpec((1, HID), lambda i: (0, 0)),
        out_shape=jax.ShapeDtypeStruct((1, HID), jnp.float32),
        scratch_shapes=[pltpu.VMEM((1, HID), jnp.float32)],
        compiler_params=pltpu.CompilerParams(
            dimension_semantics=("arbitrary",)),
    )(w, h2, b2)


def _dsum_body(a_ref, o_ref):
    o_ref[...] = a_ref[0] + a_ref[1]


def _dsum(den):
    return pl.pallas_call(
        _dsum_body,
        in_specs=[pl.BlockSpec((2, 8, NP // 8), lambda: (0, 0, 0))],
        out_specs=pl.BlockSpec((8, NP // 8), lambda: (0, 0)),
        out_shape=jax.ShapeDtypeStruct((8, NP // 8), jnp.float32),
    )(den.reshape(2, 8, NP // 8)).reshape(NP)


def _att_body(s_hbm, d_hbm, src_hbm, dst_hbm, m_hbm,
              ep_hbm, den_hbm,
              s_v, d_v, src_v, dst_v, ep_v, m_v, zline_v, den_sh):
    c = lax.axis_index("c")
    t = lax.axis_index("s")
    gb = (c * NS + t) * GPT
    pltpu.sync_copy(s_hbm, s_v)
    pltpu.sync_copy(d_hbm, d_v)
    pltpu.sync_copy(m_hbm, m_v)
    pltpu.sync_copy(src_hbm.at[pl.ds(gb, GPT)], src_v)
    pltpu.sync_copy(dst_hbm.at[pl.ds(gb, GPT)], dst_v)

    zv = jnp.zeros((L,), jnp.float32)
    for q in range(640 // L):
        zline_v[pl.ds(q * L, L)] = zv
    pltpu.sync_copy(zline_v, den_sh.at[pl.ds(t * 640, 640)])
    plsc.subcore_barrier()

    mvec = m_v[...]

    def group(g, carry):
        for q in range(128 // L):
            sl = pl.ds(q * L, L)
            srcv = src_v[g, sl]
            dstv = dst_v[g, sl]
            z = plsc.load_gather(s_v, [srcv]) + plsc.load_gather(d_v, [dstv])
            e = jnp.where(z >= 0.0, z, 0.2 * z) - mvec
            ep_v[g, sl] = jnp.exp(e)
        pltpu.sync_copy(ep_v.at[g], den_sh.at[dst_v.at[g]], add=True)
        return carry

    lax.fori_loop(0, GPT, group, 0)
    pltpu.sync_copy(ep_v, ep_hbm.at[pl.ds(gb, GPT)])
    plsc.subcore_barrier()

    @pl.when(t == 0)
    def _():
        pltpu.sync_copy(den_sh, den_hbm.at[c])


def _att(s, d, src2d, dst2d, m16):
    return pl.kernel(
        _att_body,
        out_type=[
            jax.ShapeDtypeStruct((G, 128), jnp.float32),
            jax.ShapeDtypeStruct((NC, NP), jnp.float32),
        ],
        mesh=_SC_MESH,
        compiler_params=pltpu.CompilerParams(needs_layout_passes=False),
        scratch_types=[
            pltpu.VMEM((NP,), jnp.float32),
            pltpu.VMEM((NP,), jnp.float32),
            pltpu.VMEM((GPT, 128), jnp.int32),
            pltpu.VMEM((GPT, 128), jnp.int32),
            pltpu.VMEM((GPT, 128), jnp.float32),
            pltpu.VMEM((L,), jnp.float32),
            pltpu.VMEM((640,), jnp.float32),
            pltpu.VMEM_SHARED((NP,), jnp.float32),
        ],
    )(s, d, src2d, dst2d, m16)


NG2 = EPAD // NS // 128
ZR2 = 16


def _rows_body(h1a_hbm, h1b_hbm, ep_hbm, den_hbm, sd_hbm,
               o1a_hbm, o1b_hbm,
               den_v, sd_v, epg_v, dsc_v, al_v, rows_v, zb_v,
               isem0, isem1, gsem0, gsem1, ssem0, ssem1,
               acc_sh):
    c = lax.axis_index("c")
    t = lax.axis_index("s")
    isems = (isem0, isem1)
    gsems = (gsem0, gsem1)
    ssems = (ssem0, ssem1)
    gbase = t * NG2

    def start_idx(g, b):
        pltpu.async_copy(sd_hbm.at[pl.ds(gbase + g, 1)],
                         sd_v.at[pl.ds(b, 1)], isems[b])
        pltpu.async_copy(ep_hbm.at[pl.ds(gbase + g, 1)],
                         epg_v.at[pl.ds(b, 1)], isems[b])

    def wait_idx(g, b):
        pltpu.make_async_copy(sd_hbm.at[pl.ds(gbase + g, 1)],
                              sd_v.at[pl.ds(b, 1)], isems[b]).wait()
        pltpu.make_async_copy(ep_hbm.at[pl.ds(gbase + g, 1)],
                              epg_v.at[pl.ds(b, 1)], isems[b]).wait()

    def start_gather(b):
        idxref = sd_v.at[b].at[0]
        dst = rows_v.at[pl.ds(b * 128, 128)]

        @pl.when(c == 0)
        def _():
            pltpu.async_copy(h1a_hbm.at[idxref], dst, gsems[b])

        @pl.when(c == 1)
        def _():
            pltpu.async_copy(h1b_hbm.at[idxref], dst, gsems[b])

    def wait_gather(b):
        pltpu.make_async_copy(h1a_hbm.at[sd_v.at[b].at[0]],
                              rows_v.at[pl.ds(b * 128, 128)],
                              gsems[b]).wait()

    def start_scatter(b):
        pltpu.async_copy(rows_v.at[pl.ds(b * 128, 128)],
                         acc_sh.at[dsc_v.at[b]], ssems[b], add=True)

    def wait_scatter(b):
        pltpu.make_async_copy(rows_v.at[pl.ds(b * 128, 128)],
                              acc_sh.at[dsc_v.at[b]], ssems[b]).wait()

    pltpu.sync_copy(den_hbm, den_v)

    zv = jnp.zeros((L,), jnp.float32)
    for j in range(ZR2):
        for f in range(HH // L):
            zb_v[j, pl.ds(f * L, L)] = zv

    def zcp(j, carry):
        pltpu.sync_copy(zb_v, acc_sh.at[pl.ds(t * RPT + j * ZR2, ZR2)])
        return carry

    lax.fori_loop(0, RPT // ZR2, zcp, 0)
    plsc.subcore_barrier()

    start_idx(0, 0)
    start_idx(1, 1)
    wait_idx(0, 0)
    start_gather(0)

    zero16 = jnp.zeros((L,), jnp.int32)

    def pair(gp, carry):
        for b in range(2):
            g = gp * 2 + b
            nb = 1 - b
            wait_gather(b)

            @pl.when(g + 1 < NG2)
            def _():
                wait_idx(g + 1, nb)

                @pl.when(g >= 1)
                def _():
                    wait_scatter(nb)

                start_gather(nb)

            for q in range(128 // L):
                sl = pl.ds(q * L, L)
                dstv = sd_v[b, 1, sl]
                den = plsc.load_gather(den_v, [dstv]) + 1e-16
                al_v[sl] = epg_v[b, sl] / den
                dsc_v[b, sl] = dstv

            @pl.when(g + 2 < NG2)
            def _():
                start_idx(g + 2, b)

            def scale(j, carry2):
                av = plsc.load_gather(al_v, [zero16 + j])
                for f in range(HH // L):
                    slf = pl.ds(f * L, L)
                    rows_v[b * 128 + j, slf] = rows_v[b * 128 + j, slf] * av
                return carry2

            lax.fori_loop(0, 128, scale, 0)
            start_scatter(b)
        return carry

    lax.fori_loop(0, NG2 // 2, pair, 0)
    wait_scatter(0)
    wait_scatter(1)
    plsc.subcore_barrier()

    rsl = pl.ds(t * RPT, RPT)

    @pl.when(c == 0)
    def _():
        pltpu.sync_copy(acc_sh.at[rsl], o1a_hbm.at[rsl])

    @pl.when(c == 1)
    def _():
        pltpu.sync_copy(acc_sh.at[rsl], o1b_hbm.at[rsl])


def _rows(h1a, h1b, ep2d, den, sd):
    return pl.kernel(
        _rows_body,
        out_type=[
            jax.ShapeDtypeStruct((NP, HH), jnp.float32),
            jax.ShapeDtypeStruct((NP, HH), jnp.float32),
        ],
        mesh=_SC_MESH,
        compiler_params=pltpu.CompilerParams(needs_layout_passes=False),
        scratch_types=[
            pltpu.VMEM((NP,), jnp.float32),
            pltpu.VMEM((2, 2, 128), jnp.int32),
            pltpu.VMEM((2, 128), jnp.float32),
            pltpu.VMEM((2, 128), jnp.int32),
            pltpu.VMEM((128,), jnp.float32),
            pltpu.VMEM((256, HH), jnp.float32),
            pltpu.VMEM((ZR2, HH), jnp.float32),
            pltpu.SemaphoreType.DMA,
            pltpu.SemaphoreType.DMA,
            pltpu.SemaphoreType.DMA,
            pltpu.SemaphoreType.DMA,
            pltpu.SemaphoreType.DMA,
            pltpu.SemaphoreType.DMA,
            pltpu.VMEM_SHARED((NP, HH), jnp.float32),
        ],
    )(h1a, h1b, ep2d, den, sd)


def _watt_body(ep_hbm, den_hbm, src_hbm, dst_hbm, w_hbm,
               src_v, dst_v, ep_v, al_v, den_v, zline_v, w_sh):
    c = lax.axis_index("c")
    t = lax.axis_index("s")
    gb = (c * NS + t) * GPT
    pltpu.sync_copy(src_hbm.at[pl.ds(gb, GPT)], src_v)
    pltpu.sync_copy(dst_hbm.at[pl.ds(gb, GPT)], dst_v)
    pltpu.sync_copy(ep_hbm.at[pl.ds(gb, GPT)], ep_v)
    pltpu.sync_copy(den_hbm, den_v)

    zv = jnp.zeros((L,), jnp.float32)
    for q in range(640 // L):
        zline_v[pl.ds(q * L, L)] = zv
    pltpu.sync_copy(zline_v, w_sh.at[pl.ds(t * 640, 640)])
    plsc.subcore_barrier()

    def group(g, carry):
        for q in range(128 // L):
            sl = pl.ds(q * L, L)
            dstv = dst_v[g, sl]
            den = plsc.load_gather(den_v, [dstv]) + 1e-16
            al_v[g, sl] = ep_v[g, sl] / den
        pltpu.sync_copy(al_v.at[g], w_sh.at[src_v.at[g]], add=True)
        return carry

    lax.fori_loop(0, GPT, group, 0)
    plsc.subcore_barrier()

    @pl.when(t == 0)
    def _():
        pltpu.sync_copy(zline_v.at[pl.ds(0, NP - N)], w_sh.at[pl.ds(N, NP - N)])
        pltpu.sync_copy(w_sh, w_hbm.at[c])


def _watt(ep2d, den, src2d, dst2d):
    return pl.kernel(
        _watt_body,
        out_type=jax.ShapeDtypeStruct((NC, NP), jnp.float32),
        mesh=_SC_MESH,
        compiler_params=pltpu.CompilerParams(needs_layout_passes=False),
        scratch_types=[
            pltpu.VMEM((GPT, 128), jnp.int32),
            pltpu.VMEM((GPT, 128), jnp.int32),
            pltpu.VMEM((GPT, 128), jnp.float32),
            pltpu.VMEM((GPT, 128), jnp.float32),
            pltpu.VMEM((NP,), jnp.float32),
            pltpu.VMEM((640,), jnp.float32),
            pltpu.VMEM_SHARED((NP,), jnp.float32),
        ],
    )(ep2d, den, src2d, dst2d)


@jax.jit
def kernel(x, edge_index, W1, a_src1, a_dst1, b1, W2, a_src2, a_dst2, b2):
    xp = jnp.zeros((NP, IN_C), jnp.float32).at[:N, :].set(x)
    src = jnp.concatenate(
        [edge_index[0], jnp.full((EPAD - E,), PADN, jnp.int32)])
    dst = jnp.concatenate(
        [edge_index[1], jnp.full((EPAD - E,), PADN, jnp.int32)])
    src2d = src.reshape(G, 128)
    dst2d = dst.reshape(G, 128)
    sd = jnp.stack([src2d, dst2d], axis=1)

    h1s, s1, d1, sm1, dm1 = _mm1(
        xp, W1, a_src1.reshape(HID, 1), a_dst1.reshape(HID, 1))
    m1 = jnp.maximum(sm1[0, 0] + dm1[0, 0], 0.0)
    m16_1 = jnp.full((L,), m1, jnp.float32)

    ep1, den1 = _att(s1.reshape(NP), d1.reshape(NP), src2d, dst2d, m16_1)
    denc1 = _dsum(den1)

    o1a, o1b = _rows(h1s[0], h1s[1], ep1, denc1, sd)
    o1 = jnp.stack([o1a, o1b])

    h2, s2, d2, sm2, dm2 = _mm2(
        o1, b1.reshape(1, HID), W2,
        a_src2.reshape(HID, 1), a_dst2.reshape(HID, 1))
    m2 = jnp.maximum(sm2[0, 0] + dm2[0, 0], 0.0)
    m16_2 = jnp.full((L,), m2, jnp.float32)

    ep2, den2 = _att(s2.reshape(NP), d2.reshape(NP), src2d, dst2d, m16_2)
    denc2 = _dsum(den2)

    w = _watt(ep2, denc2, src2d, dst2d)

    return _pool(w, h2, b2.reshape(1, HID))


# speedup vs baseline: 1.0002x; 1.0002x over previous
import jax
import jax.numpy as jnp
from jax import lax
from jax.experimental import pallas as pl
from jax.experimental.pallas import tpu as pltpu
from jax.experimental.pallas import tpu_sc as plsc

N = 10000
E = 320000
IN_C = 128
HID = 256

NC = 2
NS = 16
L = 16

NP = 10240
PADN = 10200
EPAD = 327680
G = EPAD // 128
GPT = G // (NC * NS)
BLK = 1024
GRID = NP // BLK

HH = HID // 2
CH = 256
EPC = EPAD // NS
NCH = EPC // CH
RPT = NP // NS
ZR = 32

_SC_MESH = plsc.VectorSubcoreMesh(core_axis_name="c", subcore_axis_name="s")


def _mm1_body(x_ref, w_ref, as_ref, ad_ref,
              h_ref, s_ref, d_ref, sm_ref, dm_ref, mx_ref):
    i = pl.program_id(0)
    h = jnp.dot(x_ref[...], w_ref[...], preferred_element_type=jnp.float32)
    h_ref[0] = h[:, :HH]
    h_ref[1] = h[:, HH:]
    s = jnp.dot(h, as_ref[...], preferred_element_type=jnp.float32)
    d = jnp.dot(h, ad_ref[...], preferred_element_type=jnp.float32)
    s_ref[...] = s
    d_ref[...] = d
    sblk = jnp.max(s)
    dblk = jnp.max(d)

    @pl.when(i == 0)
    def _():
        mx_ref[0, 0] = sblk
        mx_ref[0, 1] = dblk

    @pl.when(i > 0)
    def _():
        mx_ref[0, 0] = jnp.maximum(mx_ref[0, 0], sblk)
        mx_ref[0, 1] = jnp.maximum(mx_ref[0, 1], dblk)

    @pl.when(i == GRID - 1)
    def _():
        sm_ref[...] = jnp.full((1, 1), mx_ref[0, 0], jnp.float32)
        dm_ref[...] = jnp.full((1, 1), mx_ref[0, 1], jnp.float32)


def _mm1(x, w1, a_src, a_dst):
    return pl.pallas_call(
        _mm1_body,
        grid=(GRID,),
        in_specs=[
            pl.BlockSpec((BLK, IN_C), lambda i: (i, 0)),
            pl.BlockSpec((IN_C, HID), lambda i: (0, 0)),
            pl.BlockSpec((HID, 1), lambda i: (0, 0)),
            pl.BlockSpec((HID, 1), lambda i: (0, 0)),
        ],
        out_specs=[
            pl.BlockSpec((2, BLK, HH), lambda i: (0, i, 0)),
            pl.BlockSpec((BLK, 1), lambda i: (i, 0)),
            pl.BlockSpec((BLK, 1), lambda i: (i, 0)),
            pl.BlockSpec((1, 1), lambda i: (0, 0)),
            pl.BlockSpec((1, 1), lambda i: (0, 0)),
        ],
        out_shape=[
            jax.ShapeDtypeStruct((2, NP, HH), jnp.float32),
            jax.ShapeDtypeStruct((NP, 1), jnp.float32),
            jax.ShapeDtypeStruct((NP, 1), jnp.float32),
            jax.ShapeDtypeStruct((1, 1), jnp.float32),
            jax.ShapeDtypeStruct((1, 1), jnp.float32),
        ],
        scratch_shapes=[pltpu.SMEM((1, 2), jnp.float32)],
        compiler_params=pltpu.CompilerParams(
            dimension_semantics=("arbitrary",)),
    )(x, w1, a_src, a_dst)


def _mm2_body(o1_ref, b1_ref, w2_ref, as_ref, ad_ref,
              h_ref, s_ref, d_ref, sm_ref, dm_ref, mx_ref):
    i = pl.program_id(0)
    hr0 = jnp.maximum(o1_ref[0] + b1_ref[:, :HH], 0.0)
    hr1 = jnp.maximum(o1_ref[1] + b1_ref[:, HH:], 0.0)
    h = (jnp.dot(hr0, w2_ref[:HH, :], preferred_element_type=jnp.float32)
         + jnp.dot(hr1, w2_ref[HH:, :], preferred_element_type=jnp.float32))
    h_ref[...] = h
    s = jnp.dot(h, as_ref[...], preferred_element_type=jnp.float32)
    d = jnp.dot(h, ad_ref[...], preferred_element_type=jnp.float32)
    s_ref[...] = s
    d_ref[...] = d
    sblk = jnp.max(s)
    dblk = jnp.max(d)

    @pl.when(i == 0)
    def _():
        mx_ref[0, 0] = sblk
        mx_ref[0, 1] = dblk

    @pl.when(i > 0)
    def _():
        mx_ref[0, 0] = jnp.maximum(mx_ref[0, 0], sblk)
        mx_ref[0, 1] = jnp.maximum(mx_ref[0, 1], dblk)

    @pl.when(i == GRID - 1)
    def _():
        sm_ref[...] = jnp.full((1, 1), mx_ref[0, 0], jnp.float32)
        dm_ref[...] = jnp.full((1, 1), mx_ref[0, 1], jnp.float32)


def _mm2(o1, b1, w2, a_src, a_dst):
    return pl.pallas_call(
        _mm2_body,
        grid=(GRID,),
        in_specs=[
            pl.BlockSpec((2, BLK, HH), lambda i: (0, i, 0)),
            pl.BlockSpec((1, HID), lambda i: (0, 0)),
            pl.BlockSpec((HID, HID), lambda i: (0, 0)),
            pl.BlockSpec((HID, 1), lambda i: (0, 0)),
            pl.BlockSpec((HID, 1), lambda i: (0, 0)),
        ],
        out_specs=[
            pl.BlockSpec((BLK, HID), lambda i: (i, 0)),
            pl.BlockSpec((BLK, 1), lambda i: (i, 0)),
            pl.BlockSpec((BLK, 1), lambda i: (i, 0)),
            pl.BlockSpec((1, 1), lambda i: (0, 0)),
            pl.BlockSpec((1, 1), lambda i: (0, 0)),
        ],
        out_shape=[
            jax.ShapeDtypeStruct((NP, HID), jnp.float32),
            jax.ShapeDtypeStruct((NP, 1), jnp.float32),
            jax.ShapeDtypeStruct((NP, 1), jnp.float32),
            jax.ShapeDtypeStruct((1, 1), jnp.float32),
            jax.ShapeDtypeStruct((1, 1), jnp.float32),
        ],
        scratch_shapes=[pltpu.SMEM((1, 2), jnp.float32)],
        compiler_params=pltpu.CompilerParams(
            dimension_semantics=("arbitrary",)),
    )(o1, b1, w2, a_src, a_dst)


def _pool_body(w_ref, h_ref, b2_ref, o_ref, acc_ref):
    i = pl.program_id(0)
    ws = w_ref[0:1, :] + w_ref[1:2, :]
    p = jnp.dot(ws, h_ref[...], preferred_element_type=jnp.float32)

    @pl.when(i == 0)
    def _():
        acc_ref[...] = p

    @pl.when(i > 0)
    def _():
        acc_ref[...] = acc_ref[...] + p

    @pl.when(i == GRID - 1)
    def _():
        o_ref[...] = acc_ref[...] + jnp.float32(N) * b2_ref[...]


def _pool(w, h2, b2):
    return pl.pallas_call(
        _pool_body,
        grid=(GRID,),
        in_specs=[
            pl.BlockSpec((2, BLK), lambda i: (0, i)),
            pl.BlockSpec((BLK, HID), lambda i: (i, 0)),
            pl.BlockSpec((1, HID), lambda i: (0, 0)),
        ],
        out_specs=pl.BlockSpec((1, HID), lambda i: (0, 0)),
        out_shape=jax.ShapeDtypeStruct((1, HID), jnp.float32),
        scratch_shapes=[pltpu.VMEM((1, HID), jnp.float32)],
        compiler_params=pltpu.CompilerParams(
            dimension_semantics=("arbitrary",)),
    )(w, h2, b2)


def _dsum_body(a_ref, o_ref):
    o_ref[...] = a_ref[0] + a_ref[1]


def _dsum(den):
    return pl.pallas_call(
        _dsum_body,
        in_specs=[pl.BlockSpec((2, 8, NP // 8), lambda: (0, 0, 0))],
        out_specs=pl.BlockSpec((8, NP // 8), lambda: (0, 0)),
        out_shape=jax.ShapeDtypeStruct((8, NP // 8), jnp.float32),
    )(den.reshape(2, 8, NP // 8)).reshape(NP)


def _att_body(s_hbm, d_hbm, src_hbm, dst_hbm, m_hbm,
              ep_hbm, den_hbm,
              s_v, d_v, src_v, dst_v, ep_v, m_v, zline_v, den_sh):
    c = lax.axis_index("c")
    t = lax.axis_index("s")
    gb = (c * NS + t) * GPT
    pltpu.sync_copy(s_hbm, s_v)
    pltpu.sync_copy(d_hbm, d_v)
    pltpu.sync_copy(m_hbm, m_v)
    pltpu.sync_copy(src_hbm.at[pl.ds(gb, GPT)], src_v)
    pltpu.sync_copy(dst_hbm.at[pl.ds(gb, GPT)], dst_v)

    zv = jnp.zeros((L,), jnp.float32)
    for q in range(640 // L):
        zline_v[pl.ds(q * L, L)] = zv
    pltpu.sync_copy(zline_v, den_sh.at[pl.ds(t * 640, 640)])
    plsc.subcore_barrier()

    mvec = m_v[...]

    def group(g, carry):
        for q in range(128 // L):
            sl = pl.ds(q * L, L)
            srcv = src_v[g, sl]
            dstv = dst_v[g, sl]
            z = plsc.load_gather(s_v, [srcv]) + plsc.load_gather(d_v, [dstv])
            e = jnp.where(z >= 0.0, z, 0.2 * z) - mvec
            ep_v[g, sl] = jnp.exp(e)
        pltpu.sync_copy(ep_v.at[g], den_sh.at[dst_v.at[g]], add=True)
        return carry

    lax.fori_loop(0, GPT, group, 0)
    pltpu.sync_copy(ep_v, ep_hbm.at[pl.ds(gb, GPT)])
    plsc.subcore_barrier()

    @pl.when(t == 0)
    def _():
        pltpu.sync_copy(den_sh, den_hbm.at[c])


def _att(s, d, src2d, dst2d, m16):
    return pl.kernel(
        _att_body,
        out_type=[
            jax.ShapeDtypeStruct((G, 128), jnp.float32),
            jax.ShapeDtypeStruct((NC, NP), jnp.float32),
        ],
        mesh=_SC_MESH,
        compiler_params=pltpu.CompilerParams(needs_layout_passes=False),
        scratch_types=[
            pltpu.VMEM((NP,), jnp.float32),
            pltpu.VMEM((NP,), jnp.float32),
            pltpu.VMEM((GPT, 128), jnp.int32),
            pltpu.VMEM((GPT, 128), jnp.int32),
            pltpu.VMEM((GPT, 128), jnp.float32),
            pltpu.VMEM((L,), jnp.float32),
            pltpu.VMEM((640,), jnp.float32),
            pltpu.VMEM_SHARED((NP,), jnp.float32),
        ],
    )(s, d, src2d, dst2d, m16)


NG2 = EPAD // NS // 128
ZR2 = 16


def _rows_body(h1a_hbm, h1b_hbm, ep_hbm, den_hbm, sd_hbm,
               o1a_hbm, o1b_hbm,
               den_v, sd_v, epg_v, dsc_v, al_v, rows_v, zb_v,
               isem0, isem1, gsem0, gsem1, ssem0, ssem1,
               acc_sh):
    c = lax.axis_index("c")
    t = lax.axis_index("s")
    isems = (isem0, isem1)
    gsems = (gsem0, gsem1)
    ssems = (ssem0, ssem1)
    gbase = t * NG2

    def start_idx(g, b):
        pltpu.async_copy(sd_hbm.at[pl.ds(gbase + g, 1)],
                         sd_v.at[pl.ds(b, 1)], isems[b])
        pltpu.async_copy(ep_hbm.at[pl.ds(gbase + g, 1)],
                         epg_v.at[pl.ds(b, 1)], isems[b])

    def wait_idx(g, b):
        pltpu.make_async_copy(sd_hbm.at[pl.ds(gbase + g, 1)],
                              sd_v.at[pl.ds(b, 1)], isems[b]).wait()
        pltpu.make_async_copy(ep_hbm.at[pl.ds(gbase + g, 1)],
                              epg_v.at[pl.ds(b, 1)], isems[b]).wait()

    def start_gather(b):
        for h in range(2):
            idxref = sd_v.at[b].at[0].at[pl.ds(h * 64, 64)]
            dst = rows_v.at[pl.ds(b * 128 + h * 64, 64)]

            @pl.when(c == 0)
            def _():
                pltpu.async_copy(h1a_hbm.at[idxref], dst, gsems[b])

            @pl.when(c == 1)
            def _():
                pltpu.async_copy(h1b_hbm.at[idxref], dst, gsems[b])

    def wait_gather(b):
        for h in range(2):
            pltpu.make_async_copy(
                h1a_hbm.at[sd_v.at[b].at[0].at[pl.ds(h * 64, 64)]],
                rows_v.at[pl.ds(b * 128 + h * 64, 64)],
                gsems[b]).wait()

    def start_scatter(b):
        pltpu.async_copy(rows_v.at[pl.ds(b * 128, 128)],
                         acc_sh.at[dsc_v.at[b]], ssems[b], add=True)

    def wait_scatter(b):
        pltpu.make_async_copy(rows_v.at[pl.ds(b * 128, 128)],
                              acc_sh.at[dsc_v.at[b]], ssems[b]).wait()

    pltpu.sync_copy(den_hbm, den_v)

    zv = jnp.zeros((L,), jnp.float32)
    for j in range(ZR2):
        for f in range(HH // L):
            zb_v[j, pl.ds(f * L, L)] = zv

    def zcp(j, carry):
        pltpu.sync_copy(zb_v, acc_sh.at[pl.ds(t * RPT + j * ZR2, ZR2)])
        return carry

    lax.fori_loop(0, RPT // ZR2, zcp, 0)
    plsc.subcore_barrier()

    start_idx(0, 0)
    start_idx(1, 1)
    wait_idx(0, 0)
    start_gather(0)

    zero16 = jnp.zeros((L,), jnp.int32)

    def pair(gp, carry):
        for b in range(2):
            g = gp * 2 + b
            nb = 1 - b
            wait_gather(b)

            @pl.when(g + 1 < NG2)
            def _():
                wait_idx(g + 1, nb)

                @pl.when(g >= 1)
                def _():
                    wait_scatter(nb)

                start_gather(nb)

            for q in range(128 // L):
                sl = pl.ds(q * L, L)
                dstv = sd_v[b, 1, sl]
                den = plsc.load_gather(den_v, [dstv]) + 1e-16
                al_v[sl] = epg_v[b, sl] / den
                dsc_v[b, sl] = dstv

            @pl.when(g + 2 < NG2)
            def _():
                start_idx(g + 2, b)

            def scale(j, carry2):
                av = plsc.load_gather(al_v, [zero16 + j])
                for f in range(HH // L):
                    slf = pl.ds(f * L, L)
                    rows_v[b * 128 + j, slf] = rows_v[b * 128 + j, slf] * av
                return carry2

            lax.fori_loop(0, 128, scale, 0)
            start_scatter(b)
        return carry

    lax.fori_loop(0, NG2 // 2, pair, 0)
    wait_scatter(0)
    wait_scatter(1)
    plsc.subcore_barrier()

    rsl = pl.ds(t * RPT, RPT)

    @pl.when(c == 0)
    def _():
        pltpu.sync_copy(acc_sh.at[rsl], o1a_hbm.at[rsl])

    @pl.when(c == 1)
    def _():
        pltpu.sync_copy(acc_sh.at[rsl], o1b_hbm.at[rsl])


def _rows(h1a, h1b, ep2d, den, sd):
    return pl.kernel(
        _rows_body,
        out_type=[
            jax.ShapeDtypeStruct((NP, HH), jnp.float32),
            jax.ShapeDtypeStruct((NP, HH), jnp.float32),
        ],
        mesh=_SC_MESH,
        compiler_params=pltpu.CompilerParams(needs_layout_passes=False),
        scratch_types=[
            pltpu.VMEM((NP,), jnp.float32),
            pltpu.VMEM((2, 2, 128), jnp.int32),
            pltpu.VMEM((2, 128), jnp.float32),
            pltpu.VMEM((2, 128), jnp.int32),
            pltpu.VMEM((128,), jnp.float32),
            pltpu.VMEM((256, HH), jnp.float32),
            pltpu.VMEM((ZR2, HH), jnp.float32),
            pltpu.SemaphoreType.DMA,
            pltpu.SemaphoreType.DMA,
            pltpu.SemaphoreType.DMA,
            pltpu.SemaphoreType.DMA,
            pltpu.SemaphoreType.DMA,
            pltpu.SemaphoreType.DMA,
            pltpu.VMEM_SHARED((NP, HH), jnp.float32),
        ],
    )(h1a, h1b, ep2d, den, sd)


def _watt_body(ep_hbm, den_hbm, src_hbm, dst_hbm, w_hbm,
               src_v, dst_v, ep_v, al_v, den_v, zline_v, w_sh):
    c = lax.axis_index("c")
    t = lax.axis_index("s")
    gb = (c * NS + t) * GPT
    pltpu.sync_copy(src_hbm.at[pl.ds(gb, GPT)], src_v)
    pltpu.sync_copy(dst_hbm.at[pl.ds(gb, GPT)], dst_v)
    pltpu.sync_copy(ep_hbm.at[pl.ds(gb, GPT)], ep_v)
    pltpu.sync_copy(den_hbm, den_v)

    zv = jnp.zeros((L,), jnp.float32)
    for q in range(640 // L):
        zline_v[pl.ds(q * L, L)] = zv
    pltpu.sync_copy(zline_v, w_sh.at[pl.ds(t * 640, 640)])
    plsc.subcore_barrier()

    def group(g, carry):
        for q in range(128 // L):
            sl = pl.ds(q * L, L)
            dstv = dst_v[g, sl]
            den = plsc.load_gather(den_v, [dstv]) + 1e-16
            al_v[g, sl] = ep_v[g, sl] / den
        pltpu.sync_copy(al_v.at[g], w_sh.at[src_v.at[g]], add=True)
        return carry

    lax.fori_loop(0, GPT, group, 0)
    plsc.subcore_barrier()

    @pl.when(t == 0)
    def _():
        pltpu.sync_copy(zline_v.at[pl.ds(0, NP - N)], w_sh.at[pl.ds(N, NP - N)])
        pltpu.sync_copy(w_sh, w_hbm.at[c])


def _watt(ep2d, den, src2d, dst2d):
    return pl.kernel(
        _watt_body,
        out_type=jax.ShapeDtypeStruct((NC, NP), jnp.float32),
        mesh=_SC_MESH,
        compiler_params=pltpu.CompilerParams(needs_layout_passes=False),
        scratch_types=[
            pltpu.VMEM((GPT, 128), jnp.int32),
            pltpu.VMEM((GPT, 128), jnp.int32),
            pltpu.VMEM((GPT, 128), jnp.float32),
            pltpu.VMEM((GPT, 128), jnp.float32),
            pltpu.VMEM((NP,), jnp.float32),
            pltpu.VMEM((640,), jnp.float32),
            pltpu.VMEM_SHARED((NP,), jnp.float32),
        ],
    )(ep2d, den, src2d, dst2d)


@jax.jit
def kernel(x, edge_index, W1, a_src1, a_dst1, b1, W2, a_src2, a_dst2, b2):
    xp = jnp.zeros((NP, IN_C), jnp.float32).at[:N, :].set(x)
    src = jnp.concatenate(
        [edge_index[0], jnp.full((EPAD - E,), PADN, jnp.int32)])
    dst = jnp.concatenate(
        [edge_index[1], jnp.full((EPAD - E,), PADN, jnp.int32)])
    src2d = src.reshape(G, 128)
    dst2d = dst.reshape(G, 128)
    sd = jnp.stack([src2d, dst2d], axis=1)

    h1s, s1, d1, sm1, dm1 = _mm1(
        xp, W1, a_src1.reshape(HID, 1), a_dst1.reshape(HID, 1))
    m1 = jnp.maximum(sm1[0, 0] + dm1[0, 0], 0.0)
    m16_1 = jnp.full((L,), m1, jnp.float32)

    ep1, den1 = _att(s1.reshape(NP), d1.reshape(NP), src2d, dst2d, m16_1)
    denc1 = _dsum(den1)

    o1a, o1b = _rows(h1s[0], h1s[1], ep1, denc1, sd)
    o1 = jnp.stack([o1a, o1b])

    h2, s2, d2, sm2, dm2 = _mm2(
        o1, b1.reshape(1, HID), W2,
        a_src2.reshape(HID, 1), a_dst2.reshape(HID, 1))
    m2 = jnp.maximum(sm2[0, 0] + dm2[0, 0], 0.0)
    m16_2 = jnp.full((L,), m2, jnp.float32)

    ep2, den2 = _att(s2.reshape(NP), d2.reshape(NP), src2d, dst2d, m16_2)
    denc2 = _dsum(den2)

    w = _watt(ep2, denc2, src2d, dst2d)

    return _pool(w, h2, b2.reshape(1, HID))


# speedup vs baseline: 1.1321x; 1.1319x over previous
import jax
import jax.numpy as jnp
from jax import lax
from jax.experimental import pallas as pl
from jax.experimental.pallas import tpu as pltpu
from jax.experimental.pallas import tpu_sc as plsc

N = 10000
E = 320000
IN_C = 128
HID = 256

NC = 2
NS = 16
L = 16

NP = 10240
PADN = 10200
EPAD = 327680
G = EPAD // 128
GPT = G // (NC * NS)
BLK = 1024
GRID = NP // BLK

HH = HID // 2
CH = 256
EPC = EPAD // NS
NCH = EPC // CH
RPT = NP // NS
ZR = 32

_SC_MESH = plsc.VectorSubcoreMesh(core_axis_name="c", subcore_axis_name="s")


def _mm1_body(x_ref, w_ref, as_ref, ad_ref,
              h_ref, s_ref, d_ref, sm_ref, dm_ref, mx_ref):
    i = pl.program_id(0)
    h = jnp.dot(x_ref[...], w_ref[...], preferred_element_type=jnp.float32)
    h_ref[0] = h[:, :HH].astype(jnp.bfloat16)
    h_ref[1] = h[:, HH:].astype(jnp.bfloat16)
    s = jnp.dot(h, as_ref[...], preferred_element_type=jnp.float32)
    d = jnp.dot(h, ad_ref[...], preferred_element_type=jnp.float32)
    s_ref[...] = s
    d_ref[...] = d
    sblk = jnp.max(s)
    dblk = jnp.max(d)

    @pl.when(i == 0)
    def _():
        mx_ref[0, 0] = sblk
        mx_ref[0, 1] = dblk

    @pl.when(i > 0)
    def _():
        mx_ref[0, 0] = jnp.maximum(mx_ref[0, 0], sblk)
        mx_ref[0, 1] = jnp.maximum(mx_ref[0, 1], dblk)

    @pl.when(i == GRID - 1)
    def _():
        sm_ref[...] = jnp.full((1, 1), mx_ref[0, 0], jnp.float32)
        dm_ref[...] = jnp.full((1, 1), mx_ref[0, 1], jnp.float32)


def _mm1(x, w1, a_src, a_dst):
    return pl.pallas_call(
        _mm1_body,
        grid=(GRID,),
        in_specs=[
            pl.BlockSpec((BLK, IN_C), lambda i: (i, 0)),
            pl.BlockSpec((IN_C, HID), lambda i: (0, 0)),
            pl.BlockSpec((HID, 1), lambda i: (0, 0)),
            pl.BlockSpec((HID, 1), lambda i: (0, 0)),
        ],
        out_specs=[
            pl.BlockSpec((2, BLK, HH), lambda i: (0, i, 0)),
            pl.BlockSpec((BLK, 1), lambda i: (i, 0)),
            pl.BlockSpec((BLK, 1), lambda i: (i, 0)),
            pl.BlockSpec((1, 1), lambda i: (0, 0)),
            pl.BlockSpec((1, 1), lambda i: (0, 0)),
        ],
        out_shape=[
            jax.ShapeDtypeStruct((2, NP, HH), jnp.bfloat16),
            jax.ShapeDtypeStruct((NP, 1), jnp.float32),
            jax.ShapeDtypeStruct((NP, 1), jnp.float32),
            jax.ShapeDtypeStruct((1, 1), jnp.float32),
            jax.ShapeDtypeStruct((1, 1), jnp.float32),
        ],
        scratch_shapes=[pltpu.SMEM((1, 2), jnp.float32)],
        compiler_params=pltpu.CompilerParams(
            dimension_semantics=("arbitrary",)),
    )(x, w1, a_src, a_dst)


def _mm2_body(o1_ref, b1_ref, w2_ref, as_ref, ad_ref,
              h_ref, s_ref, d_ref, sm_ref, dm_ref, mx_ref):
    i = pl.program_id(0)
    hr0 = jnp.maximum(o1_ref[0] + b1_ref[:, :HH], 0.0)
    hr1 = jnp.maximum(o1_ref[1] + b1_ref[:, HH:], 0.0)
    h = (jnp.dot(hr0, w2_ref[:HH, :], preferred_element_type=jnp.float32)
         + jnp.dot(hr1, w2_ref[HH:, :], preferred_element_type=jnp.float32))
    h_ref[...] = h
    s = jnp.dot(h, as_ref[...], preferred_element_type=jnp.float32)
    d = jnp.dot(h, ad_ref[...], preferred_element_type=jnp.float32)
    s_ref[...] = s
    d_ref[...] = d
    sblk = jnp.max(s)
    dblk = jnp.max(d)

    @pl.when(i == 0)
    def _():
        mx_ref[0, 0] = sblk
        mx_ref[0, 1] = dblk

    @pl.when(i > 0)
    def _():
        mx_ref[0, 0] = jnp.maximum(mx_ref[0, 0], sblk)
        mx_ref[0, 1] = jnp.maximum(mx_ref[0, 1], dblk)

    @pl.when(i == GRID - 1)
    def _():
        sm_ref[...] = jnp.full((1, 1), mx_ref[0, 0], jnp.float32)
        dm_ref[...] = jnp.full((1, 1), mx_ref[0, 1], jnp.float32)


def _mm2(o1, b1, w2, a_src, a_dst):
    return pl.pallas_call(
        _mm2_body,
        grid=(GRID,),
        in_specs=[
            pl.BlockSpec((2, BLK, HH), lambda i: (0, i, 0)),
            pl.BlockSpec((1, HID), lambda i: (0, 0)),
            pl.BlockSpec((HID, HID), lambda i: (0, 0)),
            pl.BlockSpec((HID, 1), lambda i: (0, 0)),
            pl.BlockSpec((HID, 1), lambda i: (0, 0)),
        ],
        out_specs=[
            pl.BlockSpec((BLK, HID), lambda i: (i, 0)),
            pl.BlockSpec((BLK, 1), lambda i: (i, 0)),
            pl.BlockSpec((BLK, 1), lambda i: (i, 0)),
            pl.BlockSpec((1, 1), lambda i: (0, 0)),
            pl.BlockSpec((1, 1), lambda i: (0, 0)),
        ],
        out_shape=[
            jax.ShapeDtypeStruct((NP, HID), jnp.float32),
            jax.ShapeDtypeStruct((NP, 1), jnp.float32),
            jax.ShapeDtypeStruct((NP, 1), jnp.float32),
            jax.ShapeDtypeStruct((1, 1), jnp.float32),
            jax.ShapeDtypeStruct((1, 1), jnp.float32),
        ],
        scratch_shapes=[pltpu.SMEM((1, 2), jnp.float32)],
        compiler_params=pltpu.CompilerParams(
            dimension_semantics=("arbitrary",)),
    )(o1, b1, w2, a_src, a_dst)


def _pool_body(w_ref, h_ref, b2_ref, o_ref, acc_ref):
    i = pl.program_id(0)
    ws = w_ref[0:1, :] + w_ref[1:2, :]
    p = jnp.dot(ws, h_ref[...], preferred_element_type=jnp.float32)

    @pl.when(i == 0)
    def _():
        acc_ref[...] = p

    @pl.when(i > 0)
    def _():
        acc_ref[...] = acc_ref[...] + p

    @pl.when(i == GRID - 1)
    def _():
        o_ref[...] = acc_ref[...] + jnp.float32(N) * b2_ref[...]


def _pool(w, h2, b2):
    return pl.pallas_call(
        _pool_body,
        grid=(GRID,),
        in_specs=[
            pl.BlockSpec((2, BLK), lambda i: (0, i)),
            pl.BlockSpec((BLK, HID), lambda i: (i, 0)),
            pl.BlockSpec((1, HID), lambda i: (0, 0)),
        ],
        out_specs=pl.BlockSpec((1, HID), lambda i: (0, 0)),
        out_shape=jax.ShapeDtypeStruct((1, HID), jnp.float32),
        scratch_shapes=[pltpu.VMEM((1, HID), jnp.float32)],
        compiler_params=pltpu.CompilerParams(
            dimension_semantics=("arbitrary",)),
    )(w, h2, b2)


def _dsum_body(a_ref, o_ref):
    o_ref[...] = a_ref[0] + a_ref[1]


def _dsum(den):
    return pl.pallas_call(
        _dsum_body,
        in_specs=[pl.BlockSpec((2, 8, NP // 8), lambda: (0, 0, 0))],
        out_specs=pl.BlockSpec((8, NP // 8), lambda: (0, 0)),
        out_shape=jax.ShapeDtypeStruct((8, NP // 8), jnp.float32),
    )(den.reshape(2, 8, NP // 8)).reshape(NP)


def _att_body(s_hbm, d_hbm, src_hbm, dst_hbm, m_hbm,
              ep_hbm, den_hbm,
              s_v, d_v, src_v, dst_v, ep_v, m_v, zline_v, den_sh):
    c = lax.axis_index("c")
    t = lax.axis_index("s")
    gb = (c * NS + t) * GPT
    pltpu.sync_copy(s_hbm, s_v)
    pltpu.sync_copy(d_hbm, d_v)
    pltpu.sync_copy(m_hbm, m_v)
    pltpu.sync_copy(src_hbm.at[pl.ds(gb, GPT)], src_v)
    pltpu.sync_copy(dst_hbm.at[pl.ds(gb, GPT)], dst_v)

    zv = jnp.zeros((L,), jnp.float32)
    for q in range(640 // L):
        zline_v[pl.ds(q * L, L)] = zv
    pltpu.sync_copy(zline_v, den_sh.at[pl.ds(t * 640, 640)])
    plsc.subcore_barrier()

    mvec = m_v[...]

    def group(g, carry):
        for q in range(128 // L):
            sl = pl.ds(q * L, L)
            srcv = src_v[g, sl]
            dstv = dst_v[g, sl]
            z = plsc.load_gather(s_v, [srcv]) + plsc.load_gather(d_v, [dstv])
            e = jnp.where(z >= 0.0, z, 0.2 * z) - mvec
            ep_v[g, sl] = jnp.exp(e)
        pltpu.sync_copy(ep_v.at[g], den_sh.at[dst_v.at[g]], add=True)
        return carry

    lax.fori_loop(0, GPT, group, 0)
    pltpu.sync_copy(ep_v, ep_hbm.at[pl.ds(gb, GPT)])
    plsc.subcore_barrier()

    @pl.when(t == 0)
    def _():
        pltpu.sync_copy(den_sh, den_hbm.at[c])


def _att(s, d, src2d, dst2d, m16):
    return pl.kernel(
        _att_body,
        out_type=[
            jax.ShapeDtypeStruct((G, 128), jnp.float32),
            jax.ShapeDtypeStruct((NC, NP), jnp.float32),
        ],
        mesh=_SC_MESH,
        compiler_params=pltpu.CompilerParams(needs_layout_passes=False),
        scratch_types=[
            pltpu.VMEM((NP,), jnp.float32),
            pltpu.VMEM((NP,), jnp.float32),
            pltpu.VMEM((GPT, 128), jnp.int32),
            pltpu.VMEM((GPT, 128), jnp.int32),
            pltpu.VMEM((GPT, 128), jnp.float32),
            pltpu.VMEM((L,), jnp.float32),
            pltpu.VMEM((640,), jnp.float32),
            pltpu.VMEM_SHARED((NP,), jnp.float32),
        ],
    )(s, d, src2d, dst2d, m16)


NG2 = EPAD // NS // 128
ZR2 = 16


def _rows_body(h1a_hbm, h1b_hbm, ep_hbm, den_hbm, sd_hbm,
               o1a_hbm, o1b_hbm,
               den_v, sd_v, epg_v, dsc_v, al_v, rows16_v, out_v, zb_v,
               isem0, isem1, gsem0, gsem1, ssem0,
               acc_sh):
    c = lax.axis_index("c")
    t = lax.axis_index("s")
    isems = (isem0, isem1)
    gsems = (gsem0, gsem1)
    gbase = t * NG2

    def start_idx(g, b):
        pltpu.async_copy(sd_hbm.at[pl.ds(gbase + g, 1)],
                         sd_v.at[pl.ds(b, 1)], isems[b])
        pltpu.async_copy(ep_hbm.at[pl.ds(gbase + g, 1)],
                         epg_v.at[pl.ds(b, 1)], isems[b])

    def wait_idx(g, b):
        pltpu.make_async_copy(sd_hbm.at[pl.ds(gbase + g, 1)],
                              sd_v.at[pl.ds(b, 1)], isems[b]).wait()
        pltpu.make_async_copy(ep_hbm.at[pl.ds(gbase + g, 1)],
                              epg_v.at[pl.ds(b, 1)], isems[b]).wait()

    def start_gather(b):
        for h in range(2):
            idxref = sd_v.at[b].at[0].at[pl.ds(h * 64, 64)]
            dst = rows16_v.at[pl.ds(b * 128 + h * 64, 64)]

            @pl.when(c == 0)
            def _():
                pltpu.async_copy(h1a_hbm.at[idxref], dst, gsems[b])

            @pl.when(c == 1)
            def _():
                pltpu.async_copy(h1b_hbm.at[idxref], dst, gsems[b])

    def wait_gather(b):
        for h in range(2):
            pltpu.make_async_copy(
                h1a_hbm.at[sd_v.at[b].at[0].at[pl.ds(h * 64, 64)]],
                rows16_v.at[pl.ds(b * 128 + h * 64, 64)],
                gsems[b]).wait()

    def start_scatter():
        pltpu.async_copy(out_v, acc_sh.at[dsc_v.at[0]], ssem0, add=True)

    def wait_scatter():
        pltpu.make_async_copy(out_v, acc_sh.at[dsc_v.at[0]],
                              ssem0).wait()

    pltpu.sync_copy(den_hbm, den_v)

    zv = jnp.zeros((L,), jnp.float32)
    for j in range(ZR2):
        for f in range(HH // L):
            zb_v[j, pl.ds(f * L, L)] = zv

    def zcp(j, carry):
        pltpu.sync_copy(zb_v, acc_sh.at[pl.ds(t * RPT + j * ZR2, ZR2)])
        return carry

    lax.fori_loop(0, RPT // ZR2, zcp, 0)
    plsc.subcore_barrier()

    start_idx(0, 0)
    start_idx(1, 1)
    wait_idx(0, 0)
    start_gather(0)

    zero16 = jnp.zeros((L,), jnp.int32)

    iota16 = lax.iota(jnp.int32, L)
    evc = iota16 * 2
    odc = evc + 1
    mhi = jnp.full((L,), -65536, jnp.int32)

    def pair(gp, carry):
        for b in range(2):
            g = gp * 2 + b
            nb = 1 - b
            wait_gather(b)

            @pl.when(g + 1 < NG2)
            def _():
                wait_idx(g + 1, nb)
                start_gather(nb)

            @pl.when(g >= 1)
            def _():
                wait_scatter()

            for q in range(128 // L):
                sl = pl.ds(q * L, L)
                dstv = sd_v[b, 1, sl]
                den = plsc.load_gather(den_v, [dstv]) + 1e-16
                al_v[sl] = epg_v[b, sl] / den
                dsc_v[0, sl] = dstv

            @pl.when(g + 2 < NG2)
            def _():
                start_idx(g + 2, b)

            def scale(j, carry2):
                av = plsc.load_gather(al_v, [zero16 + j])
                jv = zero16 + j
                for f in range(HH // 32):
                    xi = rows16_v[b * 128 + j, pl.ds(f * 16, 16)]
                    lo = plsc.bitcast(xi << 16, jnp.float32)
                    hi = plsc.bitcast(xi & mhi, jnp.float32)
                    plsc.store_scatter(out_v, [jv, evc + f * 32], lo * av)
                    plsc.store_scatter(out_v, [jv, odc + f * 32], hi * av)
                return carry2

            lax.fori_loop(0, 128, scale, 0)
            start_scatter()
        return carry

    lax.fori_loop(0, NG2 // 2, pair, 0)
    wait_scatter()
    plsc.subcore_barrier()

    rsl = pl.ds(t * RPT, RPT)

    @pl.when(c == 0)
    def _():
        pltpu.sync_copy(acc_sh.at[rsl], o1a_hbm.at[rsl])

    @pl.when(c == 1)
    def _():
        pltpu.sync_copy(acc_sh.at[rsl], o1b_hbm.at[rsl])


def _rows(h1a, h1b, ep2d, den, sd):
    return pl.kernel(
        _rows_body,
        out_type=[
            jax.ShapeDtypeStruct((NP, HH), jnp.float32),
            jax.ShapeDtypeStruct((NP, HH), jnp.float32),
        ],
        mesh=_SC_MESH,
        compiler_params=pltpu.CompilerParams(
            needs_layout_passes=False, use_tc_tiling_on_sc=False),
        scratch_types=[
            pltpu.VMEM((NP,), jnp.float32),
            pltpu.VMEM((2, 2, 128), jnp.int32),
            pltpu.VMEM((2, 128), jnp.float32),
            pltpu.VMEM((1, 128), jnp.int32),
            pltpu.VMEM((128,), jnp.float32),
            pltpu.VMEM((256, HH // 2), jnp.int32),
            pltpu.VMEM((128, HH), jnp.float32),
            pltpu.VMEM((ZR2, HH), jnp.float32),
            pltpu.SemaphoreType.DMA,
            pltpu.SemaphoreType.DMA,
            pltpu.SemaphoreType.DMA,
            pltpu.SemaphoreType.DMA,
            pltpu.SemaphoreType.DMA,
            pltpu.VMEM_SHARED((NP, HH), jnp.float32),
        ],
    )(h1a, h1b, ep2d, den, sd)


def _watt_body(ep_hbm, den_hbm, src_hbm, dst_hbm, w_hbm,
               src_v, dst_v, ep_v, al_v, den_v, zline_v, w_sh):
    c = lax.axis_index("c")
    t = lax.axis_index("s")
    gb = (c * NS + t) * GPT
    pltpu.sync_copy(src_hbm.at[pl.ds(gb, GPT)], src_v)
    pltpu.sync_copy(dst_hbm.at[pl.ds(gb, GPT)], dst_v)
    pltpu.sync_copy(ep_hbm.at[pl.ds(gb, GPT)], ep_v)
    pltpu.sync_copy(den_hbm, den_v)

    zv = jnp.zeros((L,), jnp.float32)
    for q in range(640 // L):
        zline_v[pl.ds(q * L, L)] = zv
    pltpu.sync_copy(zline_v, w_sh.at[pl.ds(t * 640, 640)])
    plsc.subcore_barrier()

    def group(g, carry):
        for q in range(128 // L):
            sl = pl.ds(q * L, L)
            dstv = dst_v[g, sl]
            den = plsc.load_gather(den_v, [dstv]) + 1e-16
            al_v[g, sl] = ep_v[g, sl] / den
        pltpu.sync_copy(al_v.at[g], w_sh.at[src_v.at[g]], add=True)
        return carry

    lax.fori_loop(0, GPT, group, 0)
    plsc.subcore_barrier()

    @pl.when(t == 0)
    def _():
        pltpu.sync_copy(zline_v.at[pl.ds(0, NP - N)], w_sh.at[pl.ds(N, NP - N)])
        pltpu.sync_copy(w_sh, w_hbm.at[c])


def _watt(ep2d, den, src2d, dst2d):
    return pl.kernel(
        _watt_body,
        out_type=jax.ShapeDtypeStruct((NC, NP), jnp.float32),
        mesh=_SC_MESH,
        compiler_params=pltpu.CompilerParams(needs_layout_passes=False),
        scratch_types=[
            pltpu.VMEM((GPT, 128), jnp.int32),
            pltpu.VMEM((GPT, 128), jnp.int32),
            pltpu.VMEM((GPT, 128), jnp.float32),
            pltpu.VMEM((GPT, 128), jnp.float32),
            pltpu.VMEM((NP,), jnp.float32),
            pltpu.VMEM((640,), jnp.float32),
            pltpu.VMEM_SHARED((NP,), jnp.float32),
        ],
    )(ep2d, den, src2d, dst2d)


@jax.jit
def kernel(x, edge_index, W1, a_src1, a_dst1, b1, W2, a_src2, a_dst2, b2):
    xp = jnp.zeros((NP, IN_C), jnp.float32).at[:N, :].set(x)
    src = jnp.concatenate(
        [edge_index[0], jnp.full((EPAD - E,), PADN, jnp.int32)])
    dst = jnp.concatenate(
        [edge_index[1], jnp.full((EPAD - E,), PADN, jnp.int32)])
    src2d = src.reshape(G, 128)
    dst2d = dst.reshape(G, 128)
    sd = jnp.stack([src2d, dst2d], axis=1)

    h1s, s1, d1, sm1, dm1 = _mm1(
        xp, W1, a_src1.reshape(HID, 1), a_dst1.reshape(HID, 1))
    m1 = jnp.maximum(sm1[0, 0] + dm1[0, 0], 0.0)
    m16_1 = jnp.full((L,), m1, jnp.float32)

    ep1, den1 = _att(s1.reshape(NP), d1.reshape(NP), src2d, dst2d, m16_1)
    denc1 = _dsum(den1)

    h1p = lax.bitcast_convert_type(
        h1s.reshape(2, NP, HH // 2, 2), jnp.int32)
    o1a, o1b = _rows(h1p[0], h1p[1], ep1, denc1, sd)
    o1 = jnp.stack([o1a, o1b])

    h2, s2, d2, sm2, dm2 = _mm2(
        o1, b1.reshape(1, HID), W2,
        a_src2.reshape(HID, 1), a_dst2.reshape(HID, 1))
    m2 = jnp.maximum(sm2[0, 0] + dm2[0, 0], 0.0)
    m16_2 = jnp.full((L,), m2, jnp.float32)

    ep2, den2 = _att(s2.reshape(NP), d2.reshape(NP), src2d, dst2d, m16_2)
    denc2 = _dsum(den2)

    w = _watt(ep2, denc2, src2d, dst2d)

    return _pool(w, h2, b2.reshape(1, HID))
